# Initial kernel scaffold; baseline (speedup 1.0000x reference)
import jax
import jax.numpy as jnp
from jax.experimental import pallas as pl


def kernel(edge_index, features, preference):
    raise NotImplementedError("write your pallas kernel here")


# speedup vs baseline: 10.9659x; 10.9659x over previous
import functools

import jax
import jax.numpy as jnp
from jax import lax
from jax.experimental import pallas as pl
from jax.experimental.pallas import tpu as pltpu
from jax.experimental.pallas import tpu_sc as plsc

NN = 10000
DD = 128
EE = 320000
CH = 128
NCH = EE // CH
NC, NS = 2, 16
NW = NC * NS
MAXI = (NCH + NW - 1) // NW
ACC_ROWS = 10240
RPT = ACC_ROWS // NS
ZR = 64
DUMMY = NN

_mesh = plsc.VectorSubcoreMesh(
    core_axis_name="c", subcore_axis_name="s", num_cores=NC, num_subcores=NS
)


def _norm_body(x_ref, o_ref):
    v = x_ref[...]
    n = jnp.sqrt(jnp.sum(v * v, axis=1, keepdims=True))
    o_ref[...] = v / jnp.maximum(n, 1e-12)


def _scale_body(x_ref, da_ref, db_ref, o_ref):
    dis = lax.rsqrt(da_ref[...] + db_ref[...])
    o_ref[...] = dis * x_ref[...]


def _combine_mid_body(aa_ref, ab_ref, da_ref, db_ref, x_ref, t_ref, y_ref):
    dis = lax.rsqrt(da_ref[...] + db_ref[...])
    h = dis * (aa_ref[...] + ab_ref[...])
    t_ref[...] = x_ref[...] + h
    y_ref[...] = dis * h


def _combine_fin_body(aa_ref, ab_ref, da_ref, db_ref, t_ref, o_ref):
    dis = lax.rsqrt(da_ref[...] + db_ref[...])
    o_ref[...] = t_ref[...] + dis * (aa_ref[...] + ab_ref[...])


_RB = 400


def _row_spec(width):
    return pl.BlockSpec((_RB, width), lambda i: (i, 0))


def _tc_norm(x):
    return pl.pallas_call(
        _norm_body,
        grid=(NN // _RB,),
        in_specs=[_row_spec(DD)],
        out_specs=_row_spec(DD),
        out_shape=jax.ShapeDtypeStruct((NN, DD), jnp.float32),
    )(x)


def _tc_scale(x, da, db):
    return pl.pallas_call(
        _scale_body,
        grid=(NN // _RB,),
        in_specs=[_row_spec(DD), _row_spec(1), _row_spec(1)],
        out_specs=_row_spec(DD),
        out_shape=jax.ShapeDtypeStruct((NN, DD), jnp.float32),
    )(x, da, db)


def _tc_combine_mid(aa, ab, da, db, x):
    return pl.pallas_call(
        _combine_mid_body,
        grid=(NN // _RB,),
        in_specs=[_row_spec(DD), _row_spec(DD), _row_spec(1), _row_spec(1),
                  _row_spec(DD)],
        out_specs=(_row_spec(DD), _row_spec(DD)),
        out_shape=(jax.ShapeDtypeStruct((NN, DD), jnp.float32),
                   jax.ShapeDtypeStruct((NN, DD), jnp.float32)),
    )(aa, ab, da, db, x)


def _tc_combine_fin(aa, ab, da, db, t):
    return pl.pallas_call(
        _combine_fin_body,
        grid=(NN // _RB,),
        in_specs=[_row_spec(DD), _row_spec(DD), _row_spec(1), _row_spec(1),
                  _row_spec(DD)],
        out_specs=_row_spec(DD),
        out_shape=jax.ShapeDtypeStruct((NN, DD), jnp.float32),
    )(aa, ab, da, db, t)


def _deg_body(row_hbm, col_hbm, deg_out, colp_out, rbuf, cbuf, vbuf, zbuf,
              deg_sh):
    c = lax.axis_index("c")
    s = lax.axis_index("s")
    wid = s * NC + c

    def _zz(i, _):
        zbuf[pl.ds(i * 16, 16)] = jnp.zeros((16,), jnp.float32)
        return 0

    lax.fori_loop(0, RPT // 16, _zz, 0)
    pltpu.sync_copy(zbuf, deg_sh.at[pl.ds(s * RPT, RPT)])
    plsc.subcore_barrier()

    def _chunk(i, _):
        k = wid + NW * i

        @pl.when(k < NCH)
        def _():
            off = k * CH
            pltpu.sync_copy(row_hbm.at[pl.ds(off, CH)], rbuf)
            pltpu.sync_copy(col_hbm.at[pl.ds(off, CH)], cbuf)

            def _vec(j, _):
                r = rbuf[pl.ds(j * 16, 16)]
                cc = cbuf[pl.ds(j * 16, 16)]
                m = r != cc
                vbuf[pl.ds(j * 16, 16)] = jnp.where(m, 1.0, 0.0)
                cbuf[pl.ds(j * 16, 16)] = jnp.where(m, cc, DUMMY)
                return 0

            lax.fori_loop(0, CH // 16, _vec, 0)
            pltpu.sync_copy(vbuf, deg_sh.at[rbuf], add=True)
            pltpu.sync_copy(cbuf, colp_out.at[pl.ds(off, CH)])

        return 0

    lax.fori_loop(0, MAXI, _chunk, 0)
    plsc.subcore_barrier()
    pltpu.sync_copy(deg_sh.at[pl.ds(s * RPT, RPT)],
                    deg_out.at[c, pl.ds(s * RPT, RPT)])


_deg_kernel = functools.partial(
    pl.kernel,
    out_type=(jax.ShapeDtypeStruct((NC, ACC_ROWS), jnp.float32),
              jax.ShapeDtypeStruct((EE,), jnp.int32)),
    mesh=_mesh,
    scratch_types=[
        pltpu.VMEM((CH,), jnp.int32),
        pltpu.VMEM((CH,), jnp.int32),
        pltpu.VMEM((CH,), jnp.float32),
        pltpu.VMEM((RPT,), jnp.float32),
        pltpu.VMEM_SHARED((ACC_ROWS,), jnp.float32),
    ],
)(_deg_body)


def _spmm_body(y_hbm, row_hbm, colp_hbm, acc_out, rbuf, cbuf, mbuf, zbuf,
               acc_sh, sem):
    c = lax.axis_index("c")
    s = lax.axis_index("s")
    wid = s * NC + c

    def _zz(i, _):
        def _zrow(j, _):
            zbuf[i, pl.ds(j * 16, 16)] = jnp.zeros((16,), jnp.float32)
            return 0

        lax.fori_loop(0, DD // 16, _zrow, 0)
        return 0

    lax.fori_loop(0, ZR, _zz, 0)

    def _zacc(k, _):
        pltpu.sync_copy(zbuf, acc_sh.at[pl.ds(s * RPT + k * ZR, ZR)])
        return 0

    lax.fori_loop(0, RPT // ZR, _zacc, 0)
    plsc.subcore_barrier()

    def _chunk(i, _):
        k = wid + NW * i

        @pl.when(k < NCH)
        def _():
            off = k * CH
            pltpu.sync_copy(row_hbm.at[pl.ds(off, CH)], rbuf)
            pltpu.sync_copy(colp_hbm.at[pl.ds(off, CH)], cbuf)
            pltpu.async_copy(y_hbm.at[rbuf], mbuf, sem).wait()
            pltpu.sync_copy(mbuf, acc_sh.at[cbuf], add=True)

        return 0

    lax.fori_loop(0, MAXI, _chunk, 0)
    plsc.subcore_barrier()

    def _exp(k, _):
        r0 = s * RPT + k * ZR
        pltpu.sync_copy(acc_sh.at[pl.ds(r0, ZR)],
                        acc_out.at[c, pl.ds(r0, ZR)])
        return 0

    lax.fori_loop(0, RPT // ZR, _exp, 0)


_spmm_kernel = functools.partial(
    pl.kernel,
    out_type=jax.ShapeDtypeStruct((NC, ACC_ROWS, DD), jnp.float32),
    mesh=_mesh,
    scratch_types=[
        pltpu.VMEM((CH,), jnp.int32),
        pltpu.VMEM((CH,), jnp.int32),
        pltpu.VMEM((CH, DD), jnp.float32),
        pltpu.VMEM((ZR, DD), jnp.float32),
        pltpu.VMEM_SHARED((ACC_ROWS, DD), jnp.float32),
        pltpu.SemaphoreType.DMA,
    ],
)(_spmm_body)


@jax.jit
def _impl(edge_index, features, preference):
    ei = edge_index.astype(jnp.int32)
    row, col = ei[0], ei[1]
    xcat = jnp.concatenate([preference.astype(jnp.float32),
                            features.astype(jnp.float32)], axis=0)
    x = _tc_norm(xcat)
    deg_parts, colp = _deg_kernel(row, col)
    da = deg_parts[0, :NN].reshape(NN, 1)
    db = deg_parts[1, :NN].reshape(NN, 1)
    y1 = _tc_scale(x, da, db)
    acc1 = _spmm_kernel(y1, row, colp)
    t1, y2 = _tc_combine_mid(acc1[0, :NN], acc1[1, :NN], da, db, x)
    acc2 = _spmm_kernel(y2, row, colp)
    total = _tc_combine_fin(acc2[0, :NN], acc2[1, :NN], da, db, t1)
    return total, preference


def kernel(edge_index, features, preference):
    return _impl(edge_index, features, preference)


# speedup vs baseline: 15.2656x; 1.3921x over previous
import functools

import jax
import jax.numpy as jnp
from jax import lax
from jax.experimental import pallas as pl
from jax.experimental.pallas import tpu as pltpu
from jax.experimental.pallas import tpu_sc as plsc

NN = 10000
DD = 128
EE = 320000
CH = 128
NCH = EE // CH
NC, NS = 2, 16
NW = NC * NS
MAXI = (NCH + NW - 1) // NW
ACC_ROWS = 10240
RPT = ACC_ROWS // NS
ZR = 64
DUMMY = NN

_mesh = plsc.VectorSubcoreMesh(
    core_axis_name="c", subcore_axis_name="s", num_cores=NC, num_subcores=NS
)


def _norm_body(x_ref, o_ref):
    v = x_ref[...]
    n = jnp.sqrt(jnp.sum(v * v, axis=1, keepdims=True))
    o_ref[...] = v / jnp.maximum(n, 1e-12)


def _scale_body(x_ref, da_ref, db_ref, o_ref):
    dis = lax.rsqrt(da_ref[...] + db_ref[...])
    o_ref[...] = dis * x_ref[...]


def _combine_mid_body(aa_ref, ab_ref, da_ref, db_ref, x_ref, t_ref, y_ref):
    dis = lax.rsqrt(da_ref[...] + db_ref[...])
    h = dis * (aa_ref[...] + ab_ref[...])
    t_ref[...] = x_ref[...] + h
    y_ref[...] = dis * h


def _combine_fin_body(aa_ref, ab_ref, da_ref, db_ref, t_ref, o_ref):
    dis = lax.rsqrt(da_ref[...] + db_ref[...])
    o_ref[...] = t_ref[...] + dis * (aa_ref[...] + ab_ref[...])


_RB = 400


def _row_spec(width):
    return pl.BlockSpec((_RB, width), lambda i: (i, 0))


def _tc_norm(x):
    return pl.pallas_call(
        _norm_body,
        grid=(NN // _RB,),
        in_specs=[_row_spec(DD)],
        out_specs=_row_spec(DD),
        out_shape=jax.ShapeDtypeStruct((NN, DD), jnp.float32),
    )(x)


def _tc_scale(x, da, db):
    return pl.pallas_call(
        _scale_body,
        grid=(NN // _RB,),
        in_specs=[_row_spec(DD), _row_spec(1), _row_spec(1)],
        out_specs=_row_spec(DD),
        out_shape=jax.ShapeDtypeStruct((NN, DD), jnp.float32),
    )(x, da, db)


def _tc_combine_mid(aa, ab, da, db, x):
    return pl.pallas_call(
        _combine_mid_body,
        grid=(NN // _RB,),
        in_specs=[_row_spec(DD), _row_spec(DD), _row_spec(1), _row_spec(1),
                  _row_spec(DD)],
        out_specs=(_row_spec(DD), _row_spec(DD)),
        out_shape=(jax.ShapeDtypeStruct((NN, DD), jnp.float32),
                   jax.ShapeDtypeStruct((NN, DD), jnp.float32)),
    )(aa, ab, da, db, x)


def _tc_combine_fin(aa, ab, da, db, t):
    return pl.pallas_call(
        _combine_fin_body,
        grid=(NN // _RB,),
        in_specs=[_row_spec(DD), _row_spec(DD), _row_spec(1), _row_spec(1),
                  _row_spec(DD)],
        out_specs=_row_spec(DD),
        out_shape=jax.ShapeDtypeStruct((NN, DD), jnp.float32),
    )(aa, ab, da, db, t)


def _deg_body(row_hbm, col_hbm, deg_out, colp_out, rbuf, cbuf, vbuf, zbuf,
              deg_sh):
    c = lax.axis_index("c")
    s = lax.axis_index("s")
    wid = s * NC + c

    def _zz(i, _):
        zbuf[pl.ds(i * 16, 16)] = jnp.zeros((16,), jnp.float32)
        return 0

    lax.fori_loop(0, RPT // 16, _zz, 0)
    pltpu.sync_copy(zbuf, deg_sh.at[pl.ds(s * RPT, RPT)])
    plsc.subcore_barrier()

    def _chunk(i, _):
        k = wid + NW * i

        @pl.when(k < NCH)
        def _():
            off = k * CH
            pltpu.sync_copy(row_hbm.at[pl.ds(off, CH)], rbuf)
            pltpu.sync_copy(col_hbm.at[pl.ds(off, CH)], cbuf)

            def _vec(j, _):
                r = rbuf[pl.ds(j * 16, 16)]
                cc = cbuf[pl.ds(j * 16, 16)]
                m = r != cc
                vbuf[pl.ds(j * 16, 16)] = jnp.where(m, 1.0, 0.0)
                cbuf[pl.ds(j * 16, 16)] = jnp.where(m, cc, DUMMY)
                return 0

            lax.fori_loop(0, CH // 16, _vec, 0)
            pltpu.sync_copy(vbuf, deg_sh.at[rbuf], add=True)
            pltpu.sync_copy(cbuf, colp_out.at[pl.ds(off, CH)])

        return 0

    lax.fori_loop(0, MAXI, _chunk, 0)
    plsc.subcore_barrier()
    pltpu.sync_copy(deg_sh.at[pl.ds(s * RPT, RPT)],
                    deg_out.at[c, pl.ds(s * RPT, RPT)])


_deg_kernel = functools.partial(
    pl.kernel,
    out_type=(jax.ShapeDtypeStruct((NC, ACC_ROWS), jnp.float32),
              jax.ShapeDtypeStruct((EE,), jnp.int32)),
    mesh=_mesh,
    scratch_types=[
        pltpu.VMEM((CH,), jnp.int32),
        pltpu.VMEM((CH,), jnp.int32),
        pltpu.VMEM((CH,), jnp.float32),
        pltpu.VMEM((RPT,), jnp.float32),
        pltpu.VMEM_SHARED((ACC_ROWS,), jnp.float32),
    ],
)(_deg_body)


_PAIRS = (MAXI + 1) // 2


def _spmm_body(y_hbm, row_hbm, colp_hbm, acc_out, r0, r1, c0, c1, m0, m1,
               zbuf, acc_sh, s0, s1):
    c = lax.axis_index("c")
    s = lax.axis_index("s")
    wid = s * NC + c

    def _zz(i, _):
        def _zrow(j, _):
            zbuf[i, pl.ds(j * 16, 16)] = jnp.zeros((16,), jnp.float32)
            return 0

        lax.fori_loop(0, DD // 16, _zrow, 0)
        return 0

    lax.fori_loop(0, ZR, _zz, 0)

    def _zacc(k, _):
        pltpu.sync_copy(zbuf, acc_sh.at[pl.ds(s * RPT + k * ZR, ZR)])
        return 0

    lax.fori_loop(0, RPT // ZR, _zacc, 0)
    plsc.subcore_barrier()

    def _issue(k, rb, cb, mb, sem):
        off = k * CH
        pltpu.sync_copy(row_hbm.at[pl.ds(off, CH)], rb)
        pltpu.sync_copy(colp_hbm.at[pl.ds(off, CH)], cb)
        pltpu.async_copy(y_hbm.at[rb], mb, sem)

    def _wait(mb, sem):
        pltpu.make_async_copy(y_hbm.at[pl.ds(0, CH)], mb, sem).wait()

    _issue(wid, r0, c0, m0, s0)
    _issue(wid + NW, r1, c1, m1, s1)

    def _pair(p, _):
        a = wid + NW * 2 * p

        @pl.when(a < NCH)
        def _():
            _wait(m0, s0)
            pltpu.sync_copy(m0, acc_sh.at[c0], add=True)

            @pl.when(a + 2 * NW < NCH)
            def _():
                _issue(a + 2 * NW, r0, c0, m0, s0)

        b = a + NW

        @pl.when(b < NCH)
        def _():
            _wait(m1, s1)
            pltpu.sync_copy(m1, acc_sh.at[c1], add=True)

            @pl.when(b + 2 * NW < NCH)
            def _():
                _issue(b + 2 * NW, r1, c1, m1, s1)

        return 0

    lax.fori_loop(0, _PAIRS, _pair, 0)
    plsc.subcore_barrier()

    def _exp(k, _):
        r0 = s * RPT + k * ZR
        pltpu.sync_copy(acc_sh.at[pl.ds(r0, ZR)],
                        acc_out.at[c, pl.ds(r0, ZR)])
        return 0

    lax.fori_loop(0, RPT // ZR, _exp, 0)


_spmm_kernel = functools.partial(
    pl.kernel,
    out_type=jax.ShapeDtypeStruct((NC, ACC_ROWS, DD), jnp.float32),
    mesh=_mesh,
    scratch_types=[
        pltpu.VMEM((CH,), jnp.int32),
        pltpu.VMEM((CH,), jnp.int32),
        pltpu.VMEM((CH,), jnp.int32),
        pltpu.VMEM((CH,), jnp.int32),
        pltpu.VMEM((CH, DD), jnp.float32),
        pltpu.VMEM((CH, DD), jnp.float32),
        pltpu.VMEM((ZR, DD), jnp.float32),
        pltpu.VMEM_SHARED((ACC_ROWS, DD), jnp.float32),
        pltpu.SemaphoreType.DMA,
        pltpu.SemaphoreType.DMA,
    ],
)(_spmm_body)


@jax.jit
def _impl(edge_index, features, preference):
    ei = edge_index.astype(jnp.int32)
    row, col = ei[0], ei[1]
    xcat = jnp.concatenate([preference.astype(jnp.float32),
                            features.astype(jnp.float32)], axis=0)
    x = _tc_norm(xcat)
    deg_parts, colp = _deg_kernel(row, col)
    da = deg_parts[0, :NN].reshape(NN, 1)
    db = deg_parts[1, :NN].reshape(NN, 1)
    y1 = _tc_scale(x, da, db)
    acc1 = _spmm_kernel(y1, row, colp)
    t1, y2 = _tc_combine_mid(acc1[0, :NN], acc1[1, :NN], da, db, x)
    acc2 = _spmm_kernel(y2, row, colp)
    total = _tc_combine_fin(acc2[0, :NN], acc2[1, :NN], da, db, t1)
    return total, preference


def kernel(edge_index, features, preference):
    return _impl(edge_index, features, preference)
